# baseline (speedup 1.0000x reference)
import functools

import jax
import jax.numpy as jnp
from jax import lax
from jax.experimental import pallas as pl
from jax.experimental.pallas import tpu as pltpu
from jax.experimental.pallas import tpu_sc as plsc

_CHUNK = 128


@functools.cache
def _sc_geometry():
    info = plsc.get_sparse_core_info()
    return info.num_cores, info.num_subcores


@functools.partial(jax.jit, static_argnums=(2, 3, 4))
def _gather_rows(table, idx2d, b_per_w, n_chunks, nc):
    V = table.shape[0]
    B = idx2d.shape[0] * idx2d.shape[1]
    mesh = plsc.VectorSubcoreMesh(core_axis_name="c", subcore_axis_name="s")

    @functools.partial(
        pl.kernel,
        mesh=mesh,
        out_type=jax.ShapeDtypeStruct((B, 2, 16), jnp.float32),
        scratch_types=[
            pltpu.VMEM((n_chunks, _CHUNK), jnp.int32),
            pltpu.VMEM((b_per_w, 2, 16), jnp.float32),
            pltpu.SemaphoreType.DMA,
        ],
        compiler_params=pltpu.CompilerParams(use_tc_tiling_on_sc=False),
    )
    def k(table_hbm, idx_hbm, out_hbm, idx_v, rows_v, sem):
        wid = lax.axis_index("s") * nc + lax.axis_index("c")
        base = wid * b_per_w
        pltpu.sync_copy(idx_hbm.at[pl.ds(wid * n_chunks, n_chunks)], idx_v)
        copies = []
        for j in range(n_chunks):
            copies.append(
                pltpu.async_copy(
                    table_hbm.at[idx_v.at[j]],
                    rows_v.at[pl.ds(j * _CHUNK, _CHUNK)],
                    sem,
                )
            )
        for c in copies:
            c.wait()
        pltpu.sync_copy(rows_v, out_hbm.at[pl.ds(base, b_per_w)])

    return k(table, idx2d)


def kernel(boxes, box_indices):
    nm, nb, two, dims = boxes.shape
    B = box_indices.shape[0]
    nc, ns = _sc_geometry()
    nw = nc * ns
    table = boxes.reshape(nb, two, dims)
    b_per_w = B // nw
    n_chunks = b_per_w // _CHUNK
    idx2d = box_indices.astype(jnp.int32).reshape(nw * n_chunks, _CHUNK)
    out = _gather_rows(table, idx2d, b_per_w, n_chunks, nc)
    return out.reshape(nm, B, two, dims)


# speedup vs baseline: 18.0927x; 18.0927x over previous
import functools

import jax
import jax.numpy as jnp
from jax import lax
from jax.experimental import pallas as pl
from jax.experimental.pallas import tpu as pltpu
from jax.experimental.pallas import tpu_sc as plsc

_G = 16
_TILE = 128


@functools.cache
def _sc_geometry():
    info = plsc.get_sparse_core_info()
    return info.num_cores, info.num_subcores


@functools.partial(jax.jit, static_argnums=(2, 3))
def _gather_cols(table_t, idx, b_per_w, nc):
    C, V = table_t.shape
    B = idx.shape[0]
    mesh = plsc.VectorSubcoreMesh(core_axis_name="c", subcore_axis_name="s")
    n_otiles = b_per_w // _TILE
    groups_per_otile = _TILE // _G

    @functools.partial(
        pl.kernel,
        mesh=mesh,
        out_type=jax.ShapeDtypeStruct((C, B), jnp.float32),
        scratch_types=[
            pltpu.VMEM((b_per_w,), jnp.int32),
            pltpu.VMEM((_G, C, _TILE), jnp.float32),
            pltpu.VMEM((C, _TILE), jnp.float32),
            pltpu.SemaphoreType.DMA,
            pltpu.SemaphoreType.DMA,
        ],
        compiler_params=pltpu.CompilerParams(needs_layout_passes=False),
    )
    def k(tab, idx_hbm, out, idx_v, tiles, obuf, gsem, osem):
        wid = lax.axis_index("s") * nc + lax.axis_index("c")
        base = wid * b_per_w
        pltpu.sync_copy(idx_hbm.at[pl.ds(base, b_per_w)], idx_v)
        iota = lax.iota(jnp.int32, 16)
        row_halves = [iota + 16 * h for h in range(C // 16)]

        @pl.loop(0, n_otiles)
        def _(t):
            for jg in range(groups_per_otile):
                vvec = idx_v[pl.ds(t * _TILE + jg * _G, _G)]
                ovec = vvec & 127
                copies = []
                for b in range(_G):
                    off = pl.multiple_of(vvec[b] & -128, _TILE)
                    copies.append(
                        pltpu.async_copy(
                            tab.at[:, pl.ds(off, _TILE)], tiles.at[b], gsem
                        )
                    )
                for c in copies:
                    c.wait()
                for b in range(_G):
                    col = jnp.broadcast_to(ovec[b], (16,))
                    dst_col = jnp.broadcast_to(jnp.int32(jg * _G + b), (16,))
                    for rows in row_halves:
                        vals = plsc.load_gather(tiles.at[b], [rows, col])
                        plsc.store_scatter(obuf, [rows, dst_col], vals)
            ocol = pl.multiple_of(base + t * _TILE, _TILE)
            pltpu.async_copy(obuf, out.at[:, pl.ds(ocol, _TILE)], osem).wait()

    return k(table_t, idx)


def kernel(boxes, box_indices):
    nm, nb, two, dims = boxes.shape
    C = two * dims
    B = box_indices.shape[0]
    nc, ns = _sc_geometry()
    nw = nc * ns
    b_per_w = B // nw
    table_t = boxes.reshape(nb, C).T
    idx = box_indices.astype(jnp.int32)
    out_t = _gather_cols(table_t, idx, b_per_w, nc)
    return out_t.reshape(nm, two, dims, B).transpose(0, 3, 1, 2)


# speedup vs baseline: 19.9100x; 1.1004x over previous
import functools

import jax
import jax.numpy as jnp
from jax import lax
from jax.experimental import pallas as pl
from jax.experimental.pallas import tpu as pltpu
from jax.experimental.pallas import tpu_sc as plsc

_G = 8
_TILE = 128


@functools.cache
def _sc_geometry():
    info = plsc.get_sparse_core_info()
    return info.num_cores, info.num_subcores


@functools.partial(jax.jit, static_argnums=(2, 3))
def _gather_cols(table_t, idx, b_per_w, nc):
    C, V = table_t.shape
    B = idx.shape[0]
    mesh = plsc.VectorSubcoreMesh(core_axis_name="c", subcore_axis_name="s")
    ngroups = b_per_w // _G

    @functools.partial(
        pl.kernel,
        mesh=mesh,
        out_type=jax.ShapeDtypeStruct((C, B), jnp.float32),
        scratch_types=[
            pltpu.VMEM((b_per_w + 16,), jnp.int32),
            pltpu.VMEM((2 * _G, C, _TILE), jnp.float32),
            pltpu.VMEM((C, b_per_w), jnp.float32),
            pltpu.SemaphoreType.DMA,
            pltpu.SemaphoreType.DMA,
            pltpu.SemaphoreType.DMA,
        ],
        compiler_params=pltpu.CompilerParams(needs_layout_passes=False),
    )
    def k(tab, idx_hbm, out, idx_v, tiles, obuf, sem0, sem1, osem):
        wid = lax.axis_index("s") * nc + lax.axis_index("c")
        base = wid * b_per_w
        pltpu.sync_copy(idx_hbm.at[pl.ds(base, b_per_w)], idx_v.at[pl.ds(0, b_per_w)])
        iota = lax.iota(jnp.int32, 16)
        row_halves = [iota + 16 * h for h in range(C // 16)]
        sems = (sem0, sem1)

        def fetch(g, phase, sem):
            vvec = idx_v[pl.ds(g * _G, 16)]
            for b in range(_G):
                off = pl.multiple_of(vvec[b] & -128, _TILE)
                pltpu.async_copy(
                    tab.at[:, pl.ds(off, _TILE)], tiles.at[phase * _G + b], sem
                )

        def drain(phase, sem):
            for b in range(_G):
                pltpu.make_async_copy(
                    tab.at[:, pl.ds(0, _TILE)], tiles.at[phase * _G + b], sem
                ).wait()

        def extract(g, phase):
            vvec = idx_v[pl.ds(g * _G, 16)]
            ovec = vvec & 127
            gb = g * _G
            for b in range(_G):
                col = jnp.broadcast_to(ovec[b], (16,))
                dst_col = jnp.broadcast_to((gb + b).astype(jnp.int32), (16,))
                for rows in row_halves:
                    vals = plsc.load_gather(tiles.at[phase * _G + b], [rows, col])
                    plsc.store_scatter(obuf, [rows, dst_col], vals)

        fetch(jnp.int32(0), 0, sem0)

        @pl.loop(0, ngroups // 2)
        def _(p):
            for phase in range(2):
                g = p * 2 + phase
                nsem = sems[1 - phase]

                @pl.when(g + 1 < ngroups)
                def _():
                    fetch(g + 1, 1 - phase, nsem)

                drain(phase, sems[phase])
                extract(g, phase)

        for t in range(b_per_w // _TILE):
            ocol = pl.multiple_of(base + t * _TILE, _TILE)
            pltpu.async_copy(
                obuf.at[:, pl.ds(t * _TILE, _TILE)],
                out.at[:, pl.ds(ocol, _TILE)],
                osem,
            ).wait()

    return k(table_t, idx)


def kernel(boxes, box_indices):
    nm, nb, two, dims = boxes.shape
    C = two * dims
    B = box_indices.shape[0]
    nc, ns = _sc_geometry()
    nw = nc * ns
    b_per_w = B // nw
    table_t = boxes.reshape(nb, C).T
    idx = box_indices.astype(jnp.int32)
    out_t = _gather_cols(table_t, idx, b_per_w, nc)
    return out_t.reshape(nm, two, dims, B).transpose(0, 3, 1, 2)
